# baseline (speedup 1.0000x reference)
import functools

import jax
import jax.numpy as jnp
from jax import lax
from jax.experimental import pallas as pl
from jax.experimental.pallas import tpu as pltpu
from jax.experimental.pallas import tpu_sc as plsc

_N = 10000
_E = 320000
_NC = 2
_NS = 16
_NW = _NC * _NS
_EPT = 10240
_EPAD = _NW * _EPT
_RPT = _N // _NS
_NDEG = 10240
_RPTD = _NDEG // _NS

_DOUT = [100, 70, 40, 20, 1]
_P = [112, 80, 48, 32, 16]
_CHUNK = 128
_KTOT = _EPT // _CHUNK
_CFG = {112: (4, 156), 80: (5, 155), 48: (8, 152), 32: (8, 152), 16: (8, 152)}


def _deg_call(dstp, wp, zeros_row):
    nchunk = _EPT // 128
    mesh = plsc.VectorSubcoreMesh(core_axis_name="c", subcore_axis_name="s")

    @functools.partial(
        pl.kernel,
        out_type=jax.ShapeDtypeStruct((_NC, _NDEG), jnp.float32),
        mesh=mesh,
        compiler_params=pltpu.CompilerParams(use_tc_tiling_on_sc=False),
        scratch_types=[
            pltpu.VMEM((nchunk, 128), jnp.int32),
            pltpu.VMEM((_EPT,), jnp.float32),
            pltpu.VMEM_SHARED((_NDEG,), jnp.float32),
            pltpu.SemaphoreType.DMA,
            pltpu.SemaphoreType.DMA,
        ],
    )
    def deg_k(dst_hbm, w_hbm, z_hbm, out_hbm, dst_t, w_t, acc, zsem, sem):
        c = lax.axis_index("c")
        s = lax.axis_index("s")
        wid = c * _NS + s
        zd = pltpu.async_copy(z_hbm.at[c * _NS + s],
                              acc.at[pl.ds(s * _RPTD, _RPTD)], zsem)
        pltpu.sync_copy(dst_hbm.at[wid], dst_t)
        pltpu.sync_copy(w_hbm.at[wid], w_t)
        zd.wait()
        plsc.subcore_barrier()

        def body(ci, carry):
            pltpu.async_copy(w_t.at[pl.ds(ci * 128, 128)],
                             acc.at[dst_t.at[ci]], sem, add=True)
            return carry

        lax.fori_loop(0, nchunk, body, 0)

        def drain(ci, carry):
            pltpu.make_async_copy(w_t.at[pl.ds(0, 128)],
                                  acc.at[dst_t.at[0]], sem).wait()
            return carry

        lax.fori_loop(0, nchunk, drain, 0)
        plsc.subcore_barrier()
        pltpu.sync_copy(acc.at[pl.ds(s * _RPTD, _RPTD)],
                        out_hbm.at[c, pl.ds(s * _RPTD, _RPTD)])

    return deg_k(dstp, wp, zeros_row)


def _scatter_call(meta0, w0, meta1, w1, xs, zeros_rows, p):
    nv = p // 16
    nbuf, k0 = _CFG[p]
    k1 = _KTOT * 2 - k0
    pf = max(1, nbuf - 3)
    mesh = plsc.VectorSubcoreMesh(core_axis_name="c", subcore_axis_name="s")

    @functools.partial(
        pl.kernel,
        out_type=jax.ShapeDtypeStruct((_NC, _N, p), jnp.float32),
        mesh=mesh,
        compiler_params=pltpu.CompilerParams(use_tc_tiling_on_sc=False),
        scratch_types=(
            [
                pltpu.VMEM((nbuf, 2, _CHUNK), jnp.int32),
                pltpu.VMEM((nbuf, _CHUNK), jnp.float32),
            ]
            + [pltpu.VMEM((_CHUNK, p), jnp.float32) for _ in range(nbuf)]
            + [pltpu.VMEM_SHARED((_N, p), jnp.float32)]
            + [pltpu.SemaphoreType.DMA] * (3 * nbuf + 1)
        ),
    )
    def scat_k(m0_hbm, w0_hbm, m1_hbm, w1_hbm, xs_hbm, z_hbm, out_hbm,
               ring_i, ring_w, *rest):
        bufs = rest[:nbuf]
        acc = rest[nbuf]
        gsem = rest[nbuf + 1:2 * nbuf + 1]
        ssem = rest[2 * nbuf + 1:3 * nbuf + 1]
        msem = rest[3 * nbuf + 1:4 * nbuf + 1]
        zsem = rest[4 * nbuf + 1]
        c = lax.axis_index("c")
        s = lax.axis_index("s")
        kdyn = jnp.where(c == 0, k0, k1)
        zd = pltpu.async_copy(z_hbm.at[c * _NS + s],
                              acc.at[pl.ds(s * _RPT, _RPT)], zsem)

        def issue_meta(cm, slot):
            @pl.when(c == 0)
            def _():
                pltpu.async_copy(m0_hbm.at[s, cm], ring_i.at[slot],
                                 msem[slot])
                pltpu.async_copy(w0_hbm.at[s, cm], ring_w.at[slot],
                                 msem[slot])

            @pl.when(c != 0)
            def _():
                pltpu.async_copy(m1_hbm.at[s, cm], ring_i.at[slot],
                                 msem[slot])
                pltpu.async_copy(w1_hbm.at[s, cm], ring_w.at[slot],
                                 msem[slot])

        def wait_meta(slot):
            pltpu.make_async_copy(m0_hbm.at[0, 0], ring_i.at[slot],
                                  msem[slot]).wait()
            pltpu.make_async_copy(w0_hbm.at[0, 0], ring_w.at[slot],
                                  msem[slot]).wait()

        def issue_gather(cg, slot):
            pltpu.async_copy(xs_hbm.at[ring_i.at[slot, 0]], bufs[slot],
                             gsem[slot])

        for b in range(pf + 1):
            issue_meta(jnp.int32(b), b)
        for b in range(pf):
            wait_meta(b)
            issue_gather(jnp.int32(b), b)
        zd.wait()
        plsc.subcore_barrier()

        def scale(buf, slot):
            def gbody(g, carry):
                wvec = ring_w[slot, pl.ds(g * 16, 16)]
                for j in range(16):
                    jdx = jnp.full((16,), j, dtype=jnp.int32)
                    wb = wvec.at[jdx].get(mode="promise_in_bounds")
                    e = g * 16 + j
                    for v in range(nv):
                        sl = pl.ds(v * 16, 16)
                        buf[e, sl] = buf[e, sl] * wb
                return carry

            lax.fori_loop(0, _CHUNK // 16, gbody, 0)

        def body(i, carry):
            for j in range(nbuf):
                ci = i * nbuf + j
                bm = (j + pf + 1) % nbuf
                cm = ci + pf + 1

                @pl.when(jnp.logical_and(cm >= nbuf, cm < kdyn))
                def _():
                    pltpu.make_async_copy(bufs[bm], acc.at[ring_i.at[0, 1]],
                                          ssem[bm]).wait()

                @pl.when(cm < kdyn)
                def _():
                    issue_meta(cm, bm)

                bg = (j + pf) % nbuf
                cg = ci + pf

                @pl.when(cg < kdyn)
                def _():
                    wait_meta(bg)
                    issue_gather(cg, bg)

                pltpu.make_async_copy(xs_hbm.at[ring_i.at[0, 0]], bufs[j],
                                      gsem[j]).wait()
                scale(bufs[j], j)
                pltpu.async_copy(bufs[j], acc.at[ring_i.at[j, 1]], ssem[j],
                                 add=True)
            return carry

        lax.fori_loop(0, kdyn // nbuf, body, 0)
        for b in range(nbuf):
            pltpu.make_async_copy(bufs[b], acc.at[ring_i.at[0, 1]],
                                  ssem[b]).wait()
        plsc.subcore_barrier()
        pltpu.sync_copy(acc.at[pl.ds(s * _RPT, _RPT)],
                        out_hbm.at[c, pl.ds(s * _RPT, _RPT)])

    return scat_k(meta0, w0, meta1, w1, xs, zeros_rows)


_BLK = 10000


def _mm_first(deg, x, w1p, p1):
    def body(deg_r, x_r, w_r, xs_r, dis_r):
        dis = lax.rsqrt(deg_r[...] + 2.0)
        xw = jnp.dot(x_r[...], w_r[...], preferred_element_type=jnp.float32)
        xs_r[...] = dis * xw
        dis_r[...] = dis

    return pl.pallas_call(
        body,
        grid=(_N // _BLK,),
        in_specs=[
            pl.BlockSpec((_BLK, 1), lambda i: (i, 0)),
            pl.BlockSpec((_BLK, 128), lambda i: (i, 0)),
            pl.BlockSpec((128, p1), lambda i: (0, 0)),
        ],
        out_specs=[
            pl.BlockSpec((_BLK, p1), lambda i: (i, 0)),
            pl.BlockSpec((_BLK, 1), lambda i: (i, 0)),
        ],
        out_shape=[
            jax.ShapeDtypeStruct((_N, p1), jnp.float32),
            jax.ShapeDtypeStruct((_N, 1), jnp.float32),
        ],
    )(deg, x, w1p)


def _mm_mid(acc, xs, dis, bp, wpd, pin, pout):
    def body(a_r, xs_r, dis_r, b_r, w_r, o_r):
        pre = (dis_r[...] * (a_r[0] + a_r[1] + 2.0 * xs_r[...])
               + b_r[...])
        h = jnp.maximum(pre, 0.0)
        o_r[...] = dis_r[...] * jnp.dot(h, w_r[...],
                                        preferred_element_type=jnp.float32)

    return pl.pallas_call(
        body,
        grid=(_N // _BLK,),
        in_specs=[
            pl.BlockSpec((2, _BLK, pin), lambda i: (0, i, 0)),
            pl.BlockSpec((_BLK, pin), lambda i: (i, 0)),
            pl.BlockSpec((_BLK, 1), lambda i: (i, 0)),
            pl.BlockSpec((1, pin), lambda i: (0, 0)),
            pl.BlockSpec((pin, pout), lambda i: (0, 0)),
        ],
        out_specs=pl.BlockSpec((_BLK, pout), lambda i: (i, 0)),
        out_shape=jax.ShapeDtypeStruct((_N, pout), jnp.float32),
    )(acc, xs, dis, bp, wpd)


def _mm_final(acc, xs, dis, bp, pin):
    def body(a_r, xs_r, dis_r, b_r, o_r):
        o_r[...] = (dis_r[...] * (a_r[0] + a_r[1] + 2.0 * xs_r[...])
                    + b_r[...])

    return pl.pallas_call(
        body,
        grid=(_N // _BLK,),
        in_specs=[
            pl.BlockSpec((2, _BLK, pin), lambda i: (0, i, 0)),
            pl.BlockSpec((_BLK, pin), lambda i: (i, 0)),
            pl.BlockSpec((_BLK, 1), lambda i: (i, 0)),
            pl.BlockSpec((1, pin), lambda i: (0, 0)),
        ],
        out_specs=pl.BlockSpec((_BLK, pin), lambda i: (i, 0)),
        out_shape=jax.ShapeDtypeStruct((_N, pin), jnp.float32),
    )(acc, xs, dis, bp)


def _pad2(a, rows, cols):
    return jnp.pad(a, ((0, rows - a.shape[0]), (0, cols - a.shape[1])))


def kernel(x, edge_index, edge_weight, m, f,
           W1, b1, W2, b2, W3, b3, W4, b4, W5, b5):
    del m, f
    epad = _EPAD - _E
    srcf = jnp.concatenate(
        [edge_index[0], jnp.zeros((epad,), edge_index.dtype)]
    ).astype(jnp.int32)
    dstf = jnp.concatenate(
        [edge_index[1], jnp.zeros((epad,), edge_index.dtype)]
    ).astype(jnp.int32)
    wflat = jnp.concatenate(
        [edge_weight, jnp.zeros((epad,), edge_weight.dtype)]
    )
    wp = wflat.reshape(_NW, _EPT)
    dst_deg = dstf.reshape(_NW, _EPT // _CHUNK, _CHUNK)

    def build_parts(k0):
        a = _NS * k0 * _CHUNK
        k1 = 2 * _KTOT - k0
        m0 = jnp.concatenate(
            [srcf[:a].reshape(_NS, k0, 1, _CHUNK),
             dstf[:a].reshape(_NS, k0, 1, _CHUNK)], axis=2)
        m1 = jnp.concatenate(
            [srcf[a:].reshape(_NS, k1, 1, _CHUNK),
             dstf[a:].reshape(_NS, k1, 1, _CHUNK)], axis=2)
        return (m0, wflat[:a].reshape(_NS, k0, _CHUNK),
                m1, wflat[a:].reshape(_NS, k1, _CHUNK))

    parts = {k0: build_parts(k0) for k0 in {v[1] for v in _CFG.values()}}

    ws = [W1, W2, W3, W4, W5]
    bs = [b1, b2, b3, b4, b5]
    pin_list = [128] + _P[:-1]
    wpads = [_pad2(ws[i], pin_list[i], _P[i]) for i in range(5)]
    bpads = [jnp.pad(bs[i], (0, _P[i] - bs[i].shape[0])).reshape(1, _P[i])
             for i in range(5)]

    deg2 = _deg_call(dst_deg, wp, jnp.zeros((_NW, _RPTD), jnp.float32))
    deg = (deg2[0, :_N] + deg2[1, :_N]).reshape(_N, 1)

    xs, dis = _mm_first(deg, x, wpads[0], _P[0])
    for l in range(4):
        m0, w0, m1, w1 = parts[_CFG[_P[l]][1]]
        acc = _scatter_call(m0, w0, m1, w1, xs,
                            jnp.zeros((_NW, _RPT, _P[l]), jnp.float32),
                            _P[l])
        xs = _mm_mid(acc, xs, dis, bpads[l], wpads[l + 1],
                     _P[l], _P[l + 1])
    m0, w0, m1, w1 = parts[_CFG[_P[4]][1]]
    acc = _scatter_call(m0, w0, m1, w1, xs,
                        jnp.zeros((_NW, _RPT, _P[4]), jnp.float32), _P[4])
    out = _mm_final(acc, xs, dis, bpads[4], _P[4])
    return out[:, :1]


# speedup vs baseline: 1.0013x; 1.0013x over previous
import functools

import jax
import jax.numpy as jnp
from jax import lax
from jax.experimental import pallas as pl
from jax.experimental.pallas import tpu as pltpu
from jax.experimental.pallas import tpu_sc as plsc

_N = 10000
_E = 320000
_NC = 2
_NS = 16
_NW = _NC * _NS
_EPT = 10240
_EPAD = _NW * _EPT
_RPT = _N // _NS
_NDEG = 10240
_RPTD = _NDEG // _NS

_DOUT = [100, 70, 40, 20, 1]
_P = [112, 80, 48, 32, 16]
_CHUNK = 128
_KTOT = _EPT // _CHUNK
_CFG = {112: (4, 156), 80: (5, 155), 48: (8, 152), 32: (8, 152), 16: (8, 152)}


def _deg_call(dstp, wp, zeros_row):
    nchunk = _EPT // 128
    mesh = plsc.VectorSubcoreMesh(core_axis_name="c", subcore_axis_name="s")

    @functools.partial(
        pl.kernel,
        out_type=jax.ShapeDtypeStruct((_NC, _NDEG), jnp.float32),
        mesh=mesh,
        compiler_params=pltpu.CompilerParams(use_tc_tiling_on_sc=False),
        scratch_types=[
            pltpu.VMEM((nchunk, 128), jnp.int32),
            pltpu.VMEM((_EPT,), jnp.float32),
            pltpu.VMEM_SHARED((_NDEG,), jnp.float32),
            pltpu.SemaphoreType.DMA,
            pltpu.SemaphoreType.DMA,
        ],
    )
    def deg_k(dst_hbm, w_hbm, z_hbm, out_hbm, dst_t, w_t, acc, zsem, sem):
        c = lax.axis_index("c")
        s = lax.axis_index("s")
        wid = c * _NS + s
        zd = pltpu.async_copy(z_hbm.at[c * _NS + s],
                              acc.at[pl.ds(s * _RPTD, _RPTD)], zsem)
        pltpu.sync_copy(dst_hbm.at[wid], dst_t)
        pltpu.sync_copy(w_hbm.at[wid], w_t)
        zd.wait()
        plsc.subcore_barrier()

        def body(ci, carry):
            pltpu.async_copy(w_t.at[pl.ds(ci * 128, 128)],
                             acc.at[dst_t.at[ci]], sem, add=True)
            return carry

        lax.fori_loop(0, nchunk, body, 0)

        def drain(ci, carry):
            pltpu.make_async_copy(w_t.at[pl.ds(0, 128)],
                                  acc.at[dst_t.at[0]], sem).wait()
            return carry

        lax.fori_loop(0, nchunk, drain, 0)
        plsc.subcore_barrier()
        pltpu.sync_copy(acc.at[pl.ds(s * _RPTD, _RPTD)],
                        out_hbm.at[c, pl.ds(s * _RPTD, _RPTD)])

    return deg_k(dstp, wp, zeros_row)


def _scatter_call(meta0, w0, meta1, w1, xs, zeros_rows, p):
    nv = p // 16
    nbuf, k0 = _CFG[p]
    k1 = _KTOT * 2 - k0
    pf = max(1, nbuf - 3)
    mesh = plsc.VectorSubcoreMesh(core_axis_name="c", subcore_axis_name="s")

    @functools.partial(
        pl.kernel,
        out_type=jax.ShapeDtypeStruct((_NC, _N, p), jnp.float32),
        mesh=mesh,
        compiler_params=pltpu.CompilerParams(use_tc_tiling_on_sc=False),
        scratch_types=(
            [
                pltpu.VMEM((nbuf, 2, _CHUNK), jnp.int32),
                pltpu.VMEM((nbuf, _CHUNK), jnp.float32),
            ]
            + [pltpu.VMEM((_CHUNK, p), jnp.float32) for _ in range(nbuf)]
            + [pltpu.VMEM_SHARED((_N, p), jnp.float32)]
            + [pltpu.SemaphoreType.DMA] * (3 * nbuf + 1)
        ),
    )
    def scat_k(m0_hbm, w0_hbm, m1_hbm, w1_hbm, xs_hbm, z_hbm, out_hbm,
               ring_i, ring_w, *rest):
        bufs = rest[:nbuf]
        acc = rest[nbuf]
        gsem = rest[nbuf + 1:2 * nbuf + 1]
        ssem = rest[2 * nbuf + 1:3 * nbuf + 1]
        msem = rest[3 * nbuf + 1:4 * nbuf + 1]
        zsem = rest[4 * nbuf + 1]
        c = lax.axis_index("c")
        s = lax.axis_index("s")
        kdyn = jnp.where(c == 0, k0, k1)
        zd = pltpu.async_copy(z_hbm.at[c * _NS + s],
                              acc.at[pl.ds(s * _RPT, _RPT)], zsem)

        def issue_meta(cm, slot):
            @pl.when(c == 0)
            def _():
                pltpu.async_copy(m0_hbm.at[s, cm], ring_i.at[slot],
                                 msem[slot])
                pltpu.async_copy(w0_hbm.at[s, cm], ring_w.at[slot],
                                 msem[slot])

            @pl.when(c != 0)
            def _():
                pltpu.async_copy(m1_hbm.at[s, cm], ring_i.at[slot],
                                 msem[slot])
                pltpu.async_copy(w1_hbm.at[s, cm], ring_w.at[slot],
                                 msem[slot])

        def wait_meta(slot):
            pltpu.make_async_copy(m0_hbm.at[0, 0], ring_i.at[slot],
                                  msem[slot]).wait()
            pltpu.make_async_copy(w0_hbm.at[0, 0], ring_w.at[slot],
                                  msem[slot]).wait()

        def issue_gather(cg, slot):
            pltpu.async_copy(xs_hbm.at[ring_i.at[slot, 0]], bufs[slot],
                             gsem[slot])

        for b in range(pf + 1):
            issue_meta(jnp.int32(b), b)
        for b in range(pf):
            wait_meta(b)
            issue_gather(jnp.int32(b), b)
        zd.wait()
        plsc.subcore_barrier()

        def scale(buf, slot):
            def gbody(g, carry):
                wvec = ring_w[slot, pl.ds(g * 16, 16)]
                for j in range(16):
                    jdx = jnp.full((16,), j, dtype=jnp.int32)
                    wb = wvec.at[jdx].get(mode="promise_in_bounds")
                    e = g * 16 + j
                    for v in range(nv):
                        sl = pl.ds(v * 16, 16)
                        buf[e, sl] = buf[e, sl] * wb
                return carry

            lax.fori_loop(0, _CHUNK // 16, gbody, 0)

        def body(i, carry):
            for j in range(nbuf):
                ci = i * nbuf + j
                bm = (j + pf + 1) % nbuf
                cm = ci + pf + 1

                @pl.when(jnp.logical_and(cm >= nbuf, cm < kdyn))
                def _():
                    pltpu.make_async_copy(bufs[bm], acc.at[ring_i.at[0, 1]],
                                          ssem[bm]).wait()

                @pl.when(cm < kdyn)
                def _():
                    issue_meta(cm, bm)

                bg = (j + pf) % nbuf
                cg = ci + pf

                @pl.when(cg < kdyn)
                def _():
                    wait_meta(bg)
                    issue_gather(cg, bg)

                pltpu.make_async_copy(xs_hbm.at[ring_i.at[0, 0]], bufs[j],
                                      gsem[j]).wait()
                scale(bufs[j], j)
                pltpu.async_copy(bufs[j], acc.at[ring_i.at[j, 1]], ssem[j],
                                 add=True)
            return carry

        lax.fori_loop(0, kdyn // nbuf, body, 0)
        for b in range(nbuf):
            pltpu.make_async_copy(bufs[b], acc.at[ring_i.at[0, 1]],
                                  ssem[b]).wait()
        plsc.subcore_barrier()
        pltpu.sync_copy(acc.at[pl.ds(s * _RPT, _RPT)],
                        out_hbm.at[c, pl.ds(s * _RPT, _RPT)])

    return scat_k(meta0, w0, meta1, w1, xs, zeros_rows)


_BLK = 2000


def _mm_first(deg, x, w1p, p1):
    def body(deg_r, x_r, w_r, xs_r, dis_r):
        dis = lax.rsqrt(deg_r[...] + 2.0)
        xw = jnp.dot(x_r[...], w_r[...], preferred_element_type=jnp.float32)
        xs_r[...] = dis * xw
        dis_r[...] = dis

    return pl.pallas_call(
        body,
        grid=(_N // _BLK,),
        in_specs=[
            pl.BlockSpec((_BLK, 1), lambda i: (i, 0)),
            pl.BlockSpec((_BLK, 128), lambda i: (i, 0)),
            pl.BlockSpec((128, p1), lambda i: (0, 0)),
        ],
        out_specs=[
            pl.BlockSpec((_BLK, p1), lambda i: (i, 0)),
            pl.BlockSpec((_BLK, 1), lambda i: (i, 0)),
        ],
        out_shape=[
            jax.ShapeDtypeStruct((_N, p1), jnp.float32),
            jax.ShapeDtypeStruct((_N, 1), jnp.float32),
        ],
    )(deg, x, w1p)


def _mm_mid(acc, xs, dis, bp, wpd, pin, pout):
    def body(a_r, xs_r, dis_r, b_r, w_r, o_r):
        pre = (dis_r[...] * (a_r[0] + a_r[1] + 2.0 * xs_r[...])
               + b_r[...])
        h = jnp.maximum(pre, 0.0)
        o_r[...] = dis_r[...] * jnp.dot(h, w_r[...],
                                        preferred_element_type=jnp.float32)

    return pl.pallas_call(
        body,
        grid=(_N // _BLK,),
        in_specs=[
            pl.BlockSpec((2, _BLK, pin), lambda i: (0, i, 0)),
            pl.BlockSpec((_BLK, pin), lambda i: (i, 0)),
            pl.BlockSpec((_BLK, 1), lambda i: (i, 0)),
            pl.BlockSpec((1, pin), lambda i: (0, 0)),
            pl.BlockSpec((pin, pout), lambda i: (0, 0)),
        ],
        out_specs=pl.BlockSpec((_BLK, pout), lambda i: (i, 0)),
        out_shape=jax.ShapeDtypeStruct((_N, pout), jnp.float32),
    )(acc, xs, dis, bp, wpd)


def _mm_final(acc, xs, dis, bp, pin):
    def body(a_r, xs_r, dis_r, b_r, o_r):
        o_r[...] = (dis_r[...] * (a_r[0] + a_r[1] + 2.0 * xs_r[...])
                    + b_r[...])

    return pl.pallas_call(
        body,
        grid=(_N // _BLK,),
        in_specs=[
            pl.BlockSpec((2, _BLK, pin), lambda i: (0, i, 0)),
            pl.BlockSpec((_BLK, pin), lambda i: (i, 0)),
            pl.BlockSpec((_BLK, 1), lambda i: (i, 0)),
            pl.BlockSpec((1, pin), lambda i: (0, 0)),
        ],
        out_specs=pl.BlockSpec((_BLK, pin), lambda i: (i, 0)),
        out_shape=jax.ShapeDtypeStruct((_N, pin), jnp.float32),
    )(acc, xs, dis, bp)


def _pad2(a, rows, cols):
    return jnp.pad(a, ((0, rows - a.shape[0]), (0, cols - a.shape[1])))


def kernel(x, edge_index, edge_weight, m, f,
           W1, b1, W2, b2, W3, b3, W4, b4, W5, b5):
    del m, f
    epad = _EPAD - _E
    srcf = jnp.concatenate(
        [edge_index[0], jnp.zeros((epad,), edge_index.dtype)]
    ).astype(jnp.int32)
    dstf = jnp.concatenate(
        [edge_index[1], jnp.zeros((epad,), edge_index.dtype)]
    ).astype(jnp.int32)
    wflat = jnp.concatenate(
        [edge_weight, jnp.zeros((epad,), edge_weight.dtype)]
    )
    wp = wflat.reshape(_NW, _EPT)
    dst_deg = dstf.reshape(_NW, _EPT // _CHUNK, _CHUNK)

    def build_parts(k0):
        a = _NS * k0 * _CHUNK
        k1 = 2 * _KTOT - k0
        m0 = jnp.concatenate(
            [srcf[:a].reshape(_NS, k0, 1, _CHUNK),
             dstf[:a].reshape(_NS, k0, 1, _CHUNK)], axis=2)
        m1 = jnp.concatenate(
            [srcf[a:].reshape(_NS, k1, 1, _CHUNK),
             dstf[a:].reshape(_NS, k1, 1, _CHUNK)], axis=2)
        return (m0, wflat[:a].reshape(_NS, k0, _CHUNK),
                m1, wflat[a:].reshape(_NS, k1, _CHUNK))

    parts = {k0: build_parts(k0) for k0 in {v[1] for v in _CFG.values()}}

    ws = [W1, W2, W3, W4, W5]
    bs = [b1, b2, b3, b4, b5]
    pin_list = [128] + _P[:-1]
    wpads = [_pad2(ws[i], pin_list[i], _P[i]) for i in range(5)]
    bpads = [jnp.pad(bs[i], (0, _P[i] - bs[i].shape[0])).reshape(1, _P[i])
             for i in range(5)]

    deg2 = _deg_call(dst_deg, wp, jnp.zeros((_NW, _RPTD), jnp.float32))
    deg = (deg2[0, :_N] + deg2[1, :_N]).reshape(_N, 1)

    xs, dis = _mm_first(deg, x, wpads[0], _P[0])
    for l in range(4):
        m0, w0, m1, w1 = parts[_CFG[_P[l]][1]]
        acc = _scatter_call(m0, w0, m1, w1, xs,
                            jnp.zeros((_NW, _RPT, _P[l]), jnp.float32),
                            _P[l])
        xs = _mm_mid(acc, xs, dis, bpads[l], wpads[l + 1],
                     _P[l], _P[l + 1])
    m0, w0, m1, w1 = parts[_CFG[_P[4]][1]]
    acc = _scatter_call(m0, w0, m1, w1, xs,
                        jnp.zeros((_NW, _RPT, _P[4]), jnp.float32), _P[4])
    out = _mm_final(acc, xs, dis, bpads[4], _P[4])
    return out[:, :1]
